# baseline (speedup 1.0000x reference)
import functools

import jax
import jax.numpy as jnp
from jax import lax
from jax.experimental import pallas as pl
from jax.experimental.pallas import tpu as pltpu
from jax.experimental.pallas import tpu_sc as plsc

B = 16384
NC, NS = 2, 16
NW = NC * NS
BPW = B // NW
CHUNK = 64
NCHUNK = BPW // CHUNK
DEPTH = 3
L = 16
NGRP = BPW // L

D_PROD, D_COL, D_DEPT = 21, 17, 24
D_SMALL = D_PROD + D_COL + D_DEPT
V_PROD, V_COL, V_DEPT = 133, 51, 301
OFF_P = 0
OFF_G = V_PROD * D_PROD
OFF_D = OFF_G + V_COL * D_COL
TSM = OFF_D + V_DEPT * D_DEPT


@functools.lru_cache(maxsize=None)
def _make_sc_gather(row0, nb):
  bpw = nb // NW
  nchunk = bpw // CHUNK
  ngrp = bpw // L
  mesh = plsc.VectorSubcoreMesh(core_axis_name="c", subcore_axis_name="s",
                                num_cores=NC, num_subcores=NS)

  @functools.partial(
      pl.kernel,
      out_type=(jax.ShapeDtypeStruct((nb, 256), jnp.float32),
                jax.ShapeDtypeStruct((D_SMALL, nb), jnp.float32)),
      mesh=mesh,
      compiler_params=pltpu.CompilerParams(needs_layout_passes=False),
      scratch_types=(
          [pltpu.VMEM((bpw,), jnp.int32) for _ in range(5)]
          + [pltpu.VMEM((CHUNK, 128), jnp.float32) for _ in range(2 * DEPTH)]
          + [pltpu.VMEM((TSM,), jnp.float32),
             pltpu.VMEM((D_SMALL, bpw), jnp.float32)]
          + [pltpu.SemaphoreType.DMA for _ in range(2 + 2 * DEPTH)]
      ),
  )
  def _sc_gather(cid, aid, pid, gid, did, tc, ta, tsm,
                 e12, e3t,
                 ic, ia, ip, ig, idp, rc0, rc1, rc2, ra0, ra1, ra2,
                 tsv, rst,
                 sem_s, sem_w3, gsem0, gsem1, gsem2, wsem0, wsem1, wsem2):
      w = lax.axis_index("s") * NC + lax.axis_index("c")
      base = w * bpw
      d_ic = pltpu.async_copy(cid.at[pl.ds(row0 + base, bpw)], ic, sem_s)
      d_ia = pltpu.async_copy(aid.at[pl.ds(row0 + base, bpw)], ia, sem_s)
      ds_i = [
          pltpu.async_copy(pid.at[pl.ds(row0 + base, bpw)], ip, sem_s),
          pltpu.async_copy(gid.at[pl.ds(row0 + base, bpw)], ig, sem_s),
          pltpu.async_copy(did.at[pl.ds(row0 + base, bpw)], idp, sem_s),
      ]
      dt = pltpu.async_copy(tsm, tsv, sem_s)
      d_ic.wait()
      d_ia.wait()

      cbuf = (rc0, rc1, rc2)
      abuf = (ra0, ra1, ra2)
      gsem = (gsem0, gsem1, gsem2)
      wsem = (wsem0, wsem1, wsem2)

      def fire(c):
          off = c * CHUNK
          s = gsem[c % DEPTH]
          return (
              pltpu.async_copy(tc.at[ic.at[pl.ds(off, CHUNK)]],
                               cbuf[c % DEPTH], s),
              pltpu.async_copy(ta.at[ia.at[pl.ds(off, CHUNK)]],
                               abuf[c % DEPTH], s),
          )

      gd = {c: fire(c) for c in range(DEPTH)}

      for d in ds_i:
          d.wait()
      dt.wait()

      ILP = 8

      def group_body(g, carry):
          r0 = g * L
          pb = ip[pl.ds(r0, L)] * D_PROD + OFF_P
          gb = ig[pl.ds(r0, L)] * D_COL + OFF_G
          db = idp[pl.ds(r0, L)] * D_DEPT + OFF_D
          cols = ([(d, pb + d) for d in range(D_PROD)]
                  + [(D_PROD + d, gb + d) for d in range(D_COL)]
                  + [(D_PROD + D_COL + d, db + d) for d in range(D_DEPT)])
          for i in range(0, D_SMALL, ILP):
              batch = cols[i:i + ILP]
              vals = [plsc.load_gather(tsv, [idx]) for _, idx in batch]
              for (row, _), v in zip(batch, vals):
                  rst[row, pl.ds(r0, L)] = v
          return carry

      lax.fori_loop(0, ngrp, group_body, 0)
      w3 = pltpu.async_copy(rst, e3t.at[pl.ds(0, D_SMALL), pl.ds(base, bpw)],
                            sem_w3)

      wd = {}
      for c in range(nchunk):
          gd[c][0].wait()
          gd[c][1].wait()
          r0 = base + c * CHUNK
          s = wsem[c % DEPTH]
          wd[c] = (
              pltpu.async_copy(cbuf[c % DEPTH], e12.at[pl.ds(r0, CHUNK),
                                                       pl.ds(0, 128)], s),
              pltpu.async_copy(abuf[c % DEPTH], e12.at[pl.ds(r0, CHUNK),
                                                       pl.ds(128, 128)], s),
          )
          if c + DEPTH < nchunk:
              wd[c][0].wait()
              wd[c][1].wait()
              gd[c + DEPTH] = fire(c + DEPTH)
      for c in range(max(0, nchunk - DEPTH), nchunk):
          wd[c][0].wait()
          wd[c][1].wait()
      w3.wait()

  return _sc_gather


def _mlp_body(e12, e3t, w1, b1, w2, b2, *prev_and_out):
    *prev, o_ref = prev_and_out
    h = jnp.dot(e12[...], w1[0:256], preferred_element_type=jnp.float32)
    h += lax.dot_general(e3t[...], w1[256:318], (((0,), (0,)), ((), ())),
                         preferred_element_type=jnp.float32)
    h = jnp.maximum(h + b1[...], 0.0)
    o = jnp.dot(h, w2[...], preferred_element_type=jnp.float32)
    o_ref[...] = jnp.maximum(o + b2[...], 0.0)


BT = 4096


def _mlp(e12, e3t, w1, b1, w2, b2, prev, blk0):
    nb = e12.shape[0]
    full = lambda r, c: pl.BlockSpec((r, c), lambda i: (0, 0))
    extra_specs = [] if prev is None else [pl.BlockSpec(memory_space=pl.ANY)]
    extra_args = [] if prev is None else [prev]
    def out_map(i, blk0=blk0):
        return (i + blk0, 0)
    return pl.pallas_call(
        _mlp_body,
        grid=(nb // BT,),
        in_specs=[
            pl.BlockSpec((BT, 256), lambda i: (i, 0)),
            pl.BlockSpec((D_SMALL, BT), lambda i: (0, i)),
            full(318, 256), full(1, 256), full(256, 128), full(1, 128),
        ] + extra_specs,
        out_specs=pl.BlockSpec((BT, 128), out_map),
        out_shape=jax.ShapeDtypeStruct((B, 128), jnp.float32),
        input_output_aliases={} if prev is None else {6: 0},
    )(e12, e3t, w1, b1.reshape(1, 256), w2, b2.reshape(1, 128), *extra_args)


NSPLIT = 1


def kernel(customer_id, article_id, product_type, colour_group, department,
           T_customer, T_article, T_product, T_colour, T_department,
           W1, b1, W2, b2):
    tsm = jnp.concatenate([T_product.reshape(-1), T_colour.reshape(-1),
                           T_department.reshape(-1)])
    nb = B // NSPLIT
    w2p = jnp.pad(W2, ((0, 0), (0, 64)))
    b2p = jnp.pad(b2, (0, 64))
    out = None
    for k in range(NSPLIT):
        e12, e3t = _make_sc_gather(k * nb, nb)(
            customer_id, article_id, product_type, colour_group, department,
            T_customer, T_article, tsm)
        out = _mlp(e12, e3t, W1, b1, w2p, b2p, out, k * (nb // BT))
    return out[:, 0:64]


# speedup vs baseline: 1.0034x; 1.0034x over previous
import functools

import jax
import jax.numpy as jnp
from jax import lax
from jax.experimental import pallas as pl
from jax.experimental.pallas import tpu as pltpu
from jax.experimental.pallas import tpu_sc as plsc

B = 16384
NC, NS = 2, 16
NW = NC * NS
BPW = B // NW
CHUNK = 64
NCHUNK = BPW // CHUNK
DEPTH = 3
L = 16
NGRP = BPW // L

D_PROD, D_COL, D_DEPT = 21, 17, 24
D_SMALL = D_PROD + D_COL + D_DEPT
V_PROD, V_COL, V_DEPT = 133, 51, 301
OFF_P = 0
OFF_G = V_PROD * D_PROD
OFF_D = OFF_G + V_COL * D_COL
TSM = OFF_D + V_DEPT * D_DEPT


@functools.lru_cache(maxsize=None)
def _make_sc_gather(row0, nb):
  bpw = nb // NW
  nchunk = bpw // CHUNK
  ngrp = bpw // L
  mesh = plsc.VectorSubcoreMesh(core_axis_name="c", subcore_axis_name="s",
                                num_cores=NC, num_subcores=NS)

  @functools.partial(
      pl.kernel,
      out_type=(jax.ShapeDtypeStruct((nb, 256), jnp.float32),
                jax.ShapeDtypeStruct((D_SMALL, nb), jnp.float32)),
      mesh=mesh,
      compiler_params=pltpu.CompilerParams(needs_layout_passes=False),
      scratch_types=(
          [pltpu.VMEM((bpw,), jnp.int32) for _ in range(5)]
          + [pltpu.VMEM((CHUNK, 128), jnp.float32) for _ in range(2 * DEPTH)]
          + [pltpu.VMEM((TSM,), jnp.float32),
             pltpu.VMEM((D_SMALL, bpw), jnp.float32)]
          + [pltpu.SemaphoreType.DMA for _ in range(2 + 2 * DEPTH)]
      ),
  )
  def _sc_gather(cid, aid, pid, gid, did, tc, ta, tsm,
                 e12, e3t,
                 ic, ia, ip, ig, idp, rc0, rc1, rc2, ra0, ra1, ra2,
                 tsv, rst,
                 sem_s, sem_w3, gsem0, gsem1, gsem2, wsem0, wsem1, wsem2):
      w = lax.axis_index("s") * NC + lax.axis_index("c")
      base = w * bpw
      d_ic = pltpu.async_copy(cid.at[pl.ds(row0 + base, bpw)], ic, sem_s)
      d_ia = pltpu.async_copy(aid.at[pl.ds(row0 + base, bpw)], ia, sem_s)
      ds_i = [
          pltpu.async_copy(pid.at[pl.ds(row0 + base, bpw)], ip, sem_s),
          pltpu.async_copy(gid.at[pl.ds(row0 + base, bpw)], ig, sem_s),
          pltpu.async_copy(did.at[pl.ds(row0 + base, bpw)], idp, sem_s),
      ]
      dt = pltpu.async_copy(tsm, tsv, sem_s)
      d_ic.wait()
      d_ia.wait()

      cbuf = (rc0, rc1, rc2)
      abuf = (ra0, ra1, ra2)
      gsem = (gsem0, gsem1, gsem2)
      wsem = (wsem0, wsem1, wsem2)

      def fire(c):
          off = c * CHUNK
          s = gsem[c % DEPTH]
          return (
              pltpu.async_copy(tc.at[ic.at[pl.ds(off, CHUNK)]],
                               cbuf[c % DEPTH], s),
              pltpu.async_copy(ta.at[ia.at[pl.ds(off, CHUNK)]],
                               abuf[c % DEPTH], s),
          )

      gd = {c: fire(c) for c in range(DEPTH)}

      for d in ds_i:
          d.wait()
      dt.wait()

      ILP = 8

      def group_body(g, carry):
          r0 = g * L
          pb = ip[pl.ds(r0, L)] * D_PROD + OFF_P
          gb = ig[pl.ds(r0, L)] * D_COL + OFF_G
          db = idp[pl.ds(r0, L)] * D_DEPT + OFF_D
          cols = ([(d, pb + d) for d in range(D_PROD)]
                  + [(D_PROD + d, gb + d) for d in range(D_COL)]
                  + [(D_PROD + D_COL + d, db + d) for d in range(D_DEPT)])
          for i in range(0, D_SMALL, ILP):
              batch = cols[i:i + ILP]
              vals = [plsc.load_gather(tsv, [idx]) for _, idx in batch]
              for (row, _), v in zip(batch, vals):
                  rst[row, pl.ds(r0, L)] = v
          return carry

      lax.fori_loop(0, ngrp, group_body, 0)
      w3 = pltpu.async_copy(rst, e3t.at[pl.ds(0, D_SMALL), pl.ds(base, bpw)],
                            sem_w3)

      wd = {}
      for c in range(nchunk):
          gd[c][0].wait()
          gd[c][1].wait()
          r0 = base + c * CHUNK
          s = wsem[c % DEPTH]
          wd[c] = (
              pltpu.async_copy(cbuf[c % DEPTH], e12.at[pl.ds(r0, CHUNK),
                                                       pl.ds(0, 128)], s),
              pltpu.async_copy(abuf[c % DEPTH], e12.at[pl.ds(r0, CHUNK),
                                                       pl.ds(128, 128)], s),
          )
          if c + DEPTH < nchunk:
              wd[c][0].wait()
              wd[c][1].wait()
              gd[c + DEPTH] = fire(c + DEPTH)
      for c in range(max(0, nchunk - DEPTH), nchunk):
          wd[c][0].wait()
          wd[c][1].wait()
      w3.wait()

  return _sc_gather


def _mlp_body(e12, e3t, w1, b1, w2, b2, *prev_and_out):
    *prev, o_ref = prev_and_out
    h = jnp.dot(e12[...], w1[0:256], preferred_element_type=jnp.float32)
    h += lax.dot_general(e3t[...], w1[256:318], (((0,), (0,)), ((), ())),
                         preferred_element_type=jnp.float32)
    h = jnp.maximum(h + b1[...], 0.0)
    o = jnp.dot(h, w2[...], preferred_element_type=jnp.float32)
    o_ref[...] = jnp.maximum(o + b2[...], 0.0)


BT = 4096


def _mlp(e12, e3t, w1, b1, w2, b2, prev, blk0):
    nb = e12.shape[0]
    full = lambda r, c: pl.BlockSpec((r, c), lambda i: (0, 0))
    extra_specs = [] if prev is None else [pl.BlockSpec(memory_space=pl.ANY)]
    extra_args = [] if prev is None else [prev]
    def out_map(i, blk0=blk0):
        return (i + blk0, 0)
    return pl.pallas_call(
        _mlp_body,
        grid=(nb // BT,),
        in_specs=[
            pl.BlockSpec((BT, 256), lambda i: (i, 0)),
            pl.BlockSpec((D_SMALL, BT), lambda i: (0, i)),
            full(318, 256), full(1, 256), full(256, 64), full(1, 64),
        ] + extra_specs,
        out_specs=pl.BlockSpec((BT, 64), out_map),
        out_shape=jax.ShapeDtypeStruct((B, 64), jnp.float32),
        input_output_aliases={} if prev is None else {6: 0},
    )(e12, e3t, w1, b1.reshape(1, 256), w2, b2.reshape(1, 64), *extra_args)


NSPLIT = 1


def kernel(customer_id, article_id, product_type, colour_group, department,
           T_customer, T_article, T_product, T_colour, T_department,
           W1, b1, W2, b2):
    tsm = jnp.concatenate([T_product.reshape(-1), T_colour.reshape(-1),
                           T_department.reshape(-1)])
    nb = B // NSPLIT
    out = None
    for k in range(NSPLIT):
        e12, e3t = _make_sc_gather(k * nb, nb)(
            customer_id, article_id, product_type, colour_group, department,
            T_customer, T_article, tsm)
        out = _mlp(e12, e3t, W1, b1, W2, b2, out, k * (nb // BT))
    return out
